# Initial kernel scaffold; baseline (speedup 1.0000x reference)
import numpy as np
import jax
import jax.numpy as jnp
from jax.experimental import pallas as pl
from jax.experimental.pallas import tpu as pltpu

MMD = jnp.bfloat16
BB = 128

TAP_OFFS = ((-1, -1), (-1, 0), (-1, 1),
            (0, -1),  (0, 0),  (0, 1),
            (1, -1),  (1, 0),  (1, 1))

IN_META = dict(w_off=0, w_rows=8, w_cols=3, b_col=0)
BLOCKS = (
    dict(H=16, W=16, cpi=8,  cpo=8,  proj=True,  down=False, w_off=8,
         w_rows=16, w_cols=72,  b3=1, bp=2, m=0),
    dict(H=16, W=16, cpi=8,  cpo=8,  proj=False, down=True,  w_off=24,
         w_rows=8,  w_cols=72,  b3=3, p_off=32,  m=0),
    dict(H=8,  W=8,  cpi=8,  cpo=16, proj=True,  down=False, w_off=288,
         w_rows=32, w_cols=72,  b3=4, bp=5, m=1),
    dict(H=8,  W=8,  cpi=16, cpo=16, proj=False, down=True,  w_off=320,
         w_rows=16, w_cols=144, b3=6, p_off=336, m=2),
    dict(H=4,  W=4,  cpi=16, cpo=16, proj=False, down=False, w_off=400,
         w_rows=16, w_cols=144, b3=7, m=3),
    dict(H=4,  W=4,  cpi=16, cpo=16, proj=False, down=False, w_off=416,
         w_rows=16, w_cols=144, b3=8, m=3),
)
HEAD = dict(hwf=16, cp=16, c=16, kcols=256, wbig_off=432, wsl_off=448,
            dm_off=24, bf_off=40)


def _tap_masks_tiled(H, W, bb):
    m = np.zeros((8, H * W), np.float32)
    k = 0
    for (dy, dx) in TAP_OFFS:
        if dy == 0 and dx == 0:
            continue
        r = np.arange(H)[:, None]
        c = np.arange(W)[None, :]
        ok = ((r + dy >= 0) & (r + dy < H) & (c + dx >= 0) & (c + dx < W))
        m[k] = ok.astype(np.float32).reshape(-1)
        k += 1
    return np.tile(m, (1, bb))


def _resblock(x, km, w_ref, b_ref, mask):
    H, W = km['H'], km['W']
    HW = H * W
    L = BB * HW
    cpo = km['cpo']

    taps = []
    k = 0
    for (dy, dx) in TAP_OFFS:
        if dy == 0 and dx == 0:
            taps.append(x.astype(MMD))
            continue
        shift = dy * W + dx
        rolled = pltpu.roll(x, shift=(-shift) % L, axis=1)
        taps.append((rolled * mask[k:k + 1, :]).astype(MMD))
        k += 1
    xcol = jnp.concatenate(taps, axis=0)

    Wc = w_ref[km['w_off']: km['w_off'] + km['w_rows'], 0:km['w_cols']]
    y = jnp.dot(Wc, xcol, preferred_element_type=jnp.float32)

    b3 = b_ref[0:cpo, km['b3']: km['b3'] + 1]
    if km['proj']:
        r = y[:cpo] + b3
        s = y[cpo:2 * cpo] + b_ref[0:cpo, km['bp']: km['bp'] + 1]
    else:
        r = y + b3
        s = x
    r = jnp.maximum(r, 0.2 * r)
    out = r + s

    if km['down']:
        PT = w_ref[km['p_off']: km['p_off'] + HW, 0:HW // 4]
        t = out.astype(MMD).reshape(cpo * BB, HW)
        y2 = jnp.dot(t, PT, preferred_element_type=jnp.float32)
        out = y2.reshape(cpo, BB * (HW // 4))
    return out


def _body(xt_ref, ce_ref, w_ref, b_ref, m_ref, mk0, mk1, mk2, o_ref):
    inm = IN_META
    Win = w_ref[inm['w_off']: inm['w_off'] + inm['w_rows'], 0:inm['w_cols']]
    x = (jnp.dot(Win, xt_ref[...], preferred_element_type=jnp.float32)
         + b_ref[0:inm['w_rows'], inm['b_col']: inm['b_col'] + 1])

    masks = (mk0, mk0, mk1, mk1, mk2, mk2)
    for km, mref in zip(BLOCKS, masks):
        x = _resblock(x, km, w_ref, b_ref, mref[...])

    hd = HEAD
    cp, C, HWf = hd['cp'], hd['c'], hd['hwf']
    xh = x.astype(MMD).reshape(cp * BB, HWf)
    Wbig = w_ref[hd['wbig_off']: hd['wbig_off'] + HWf, 0:hd['kcols']]
    R = jnp.dot(xh, Wbig, preferred_element_type=jnp.float32)
    Dm = m_ref[hd['dm_off']: hd['dm_off'] + cp, 0:hd['kcols']]
    s = jnp.zeros((BB, hd['kcols']), jnp.float32)
    for ci in range(cp):
        s = s + R[ci * BB:(ci + 1) * BB, :] * Dm[ci:ci + 1, :]
    Wsl = w_ref[hd['wsl_off']: hd['wsl_off'] + hd['kcols'], 0:C]
    feat = jnp.dot(s.astype(MMD), Wsl, preferred_element_type=jnp.float32)
    feat = feat + m_ref[hd['bf_off']: hd['bf_off'] + 1, 0:C]
    o_ref[...] = jnp.sum(ce_ref[...] * feat, axis=1, keepdims=True)


def kernel(wslab, bslab, mslab, emb, img, c):
    B, cimg, H0, W0 = img.shape
    HW0 = H0 * W0
    xt = (img.reshape(B, cimg, HW0).astype(MMD)
          .transpose(1, 0, 2).reshape(cimg, B * HW0))
    cemb = emb[c].astype(jnp.float32)

    mk0 = jnp.asarray(_tap_masks_tiled(16, 16, BB))
    mk1 = jnp.asarray(_tap_masks_tiled(8, 8, BB))
    mk2 = jnp.asarray(_tap_masks_tiled(4, 4, BB))

    out = pl.pallas_call(
        _body,
        out_shape=jax.ShapeDtypeStruct((B, 1), jnp.float32),
        grid=(B // BB,),
        in_specs=[
            pl.BlockSpec((cimg, BB * HW0), lambda i: (0, i)),
            pl.BlockSpec((BB, emb.shape[1]), lambda i: (i, 0)),
            pl.BlockSpec(wslab.shape, lambda i: (0, 0)),
            pl.BlockSpec(bslab.shape, lambda i: (0, 0)),
            pl.BlockSpec(mslab.shape, lambda i: (0, 0)),
            pl.BlockSpec(mk0.shape, lambda i: (0, 0)),
            pl.BlockSpec(mk1.shape, lambda i: (0, 0)),
            pl.BlockSpec(mk2.shape, lambda i: (0, 0)),
        ],
        out_specs=pl.BlockSpec((BB, 1), lambda i: (i, 0)),
        compiler_params=pltpu.CompilerParams(
            dimension_semantics=("parallel",)),
    )(xt, cemb, wslab, bslab, mslab, mk0, mk1, mk2)
    return out


# speedup vs baseline: 64.9949x; 64.9949x over previous
import numpy as np
import jax
import jax.numpy as jnp
from jax.experimental import pallas as pl
from jax.experimental.pallas import tpu as pltpu

MMD = jnp.bfloat16
BB = 128

TAP_OFFS = ((-1, -1), (-1, 0), (-1, 1),
            (0, -1),  (0, 0),  (0, 1),
            (1, -1),  (1, 0),  (1, 1))

IN_META = dict(w_off=0, w_rows=8, w_cols=3, b_col=0)
BLOCKS = (
    dict(H=16, W=16, cpi=8,  cpo=8,  proj=True,  down=False, w_off=8,
         w_rows=16, w_cols=72,  b3=1, bp=2, m=0, mstride=8),
    dict(H=16, W=16, cpi=8,  cpo=8,  proj=False, down=True,  w_off=24,
         w_rows=8,  w_cols=72,  b3=3, m=0, pool=0, mstride=8),
    dict(H=8,  W=8,  cpi=8,  cpo=16, proj=True,  down=False, w_off=288,
         w_rows=32, w_cols=72,  b3=4, bp=5, m=1, mstride=16),
    dict(H=8,  W=8,  cpi=16, cpo=16, proj=False, down=True,  w_off=320,
         w_rows=16, w_cols=144, b3=6, m=1, pool=1, mstride=16),
    dict(H=4,  W=4,  cpi=16, cpo=16, proj=False, down=False, w_off=400,
         w_rows=16, w_cols=144, b3=7, m=2, mstride=16),
    dict(H=4,  W=4,  cpi=16, cpo=16, proj=False, down=False, w_off=416,
         w_rows=16, w_cols=144, b3=8, m=2, mstride=16),
)
POOL_CFG = ({'CG': 2, 'NG': 4}, {'CG': 8, 'NG': 2})
HEAD = dict(hwf=16, cp=16, c=16, kcols=256, wbig_off=432, wsl_off=448,
            dm_off=24, bf_off=40)


def _tap_masks_pmajor(H, W, bb, ch):
    m = np.zeros((8, H * W), np.float32)
    k = 0
    for (dy, dx) in TAP_OFFS:
        if dy == 0 and dx == 0:
            continue
        r = np.arange(H)[:, None]
        c = np.arange(W)[None, :]
        ok = ((r + dy >= 0) & (r + dy < H) & (c + dx >= 0) & (c + dx < W))
        m[k] = ok.astype(np.float32).reshape(-1)
        k += 1
    m = np.repeat(m, bb, axis=1)
    return np.repeat(m, ch, axis=0)


def _pool_matrix_T(H, W):
    Ho, Wo = H // 2, W // 2
    P = np.zeros((H * W, Ho * Wo), np.float32)
    for i in range(Ho):
        for j in range(Wo):
            q = i * Wo + j
            for di in range(2):
                for dj in range(2):
                    P[(2 * i + di) * W + (2 * j + dj), q] = 0.25
    return P


def _resblock(x, km, w_ref, b_ref, mask, pools):
    H, W = km['H'], km['W']
    HW = H * W
    L = HW * BB
    cpo = km['cpo']

    cpi, mstride = km['cpi'], km['mstride']
    x16 = x.astype(MMD)
    taps = []
    k = 0
    for (dy, dx) in TAP_OFFS:
        if dy == 0 and dx == 0:
            taps.append(x16)
            continue
        sh = ((dy * W + dx) * BB) % L
        rolled = jnp.concatenate([x16[:, sh:], x16[:, :sh]], axis=1)
        taps.append(rolled * mask[k * mstride:k * mstride + cpi, :])
        k += 1
    xcol = jnp.concatenate(taps, axis=0)

    Wc = w_ref[km['w_off']: km['w_off'] + km['w_rows'], 0:km['w_cols']]
    y = jnp.dot(Wc, xcol, preferred_element_type=jnp.float32)

    b3 = b_ref[0:cpo, km['b3']: km['b3'] + 1]
    if km['proj']:
        r = y[:cpo] + b3
        s = y[cpo:2 * cpo] + b_ref[0:cpo, km['bp']: km['bp'] + 1]
    else:
        r = y + b3
        s = x
    r = jnp.maximum(r, 0.2 * r)
    out = r + s

    if 'pool' in km:
        cfg = POOL_CFG[km['pool']]
        CG, NG = cfg['CG'], cfg['NG']
        Pg = pools[km['pool']][...]
        t = out.astype(MMD).reshape(cpo * HW, BB)
        rows = CG * HW
        parts = [jnp.dot(Pg, t[g * rows:(g + 1) * rows, :],
                         preferred_element_type=jnp.float32)
                 for g in range(NG)]
        y2 = jnp.concatenate(parts, axis=0)
        out = y2.reshape(cpo, (HW // 4) * BB)
    return out


def _body(xt_ref, ce_ref, w_ref, b_ref, mk0, mk1, mk2,
          p0_ref, p1_ref, as_ref, wslt_ref, bft_ref, o_ref):
    inm = IN_META
    Win = w_ref[inm['w_off']: inm['w_off'] + inm['w_rows'], 0:inm['w_cols']]
    xin = xt_ref[...].reshape(inm['w_cols'], 256 * BB)
    x = (jnp.dot(Win, xin, preferred_element_type=jnp.float32)
         + b_ref[0:inm['w_rows'], inm['b_col']: inm['b_col'] + 1])

    masks = (mk0, mk0, mk1, mk1, mk2, mk2)
    pools = (p0_ref, p1_ref)
    for km, mref in zip(BLOCKS, masks):
        x = _resblock(x, km, w_ref, b_ref, mref[...], pools)

    hd = HEAD
    cp, C, HWf = hd['cp'], hd['c'], hd['hwf']
    xh = x.astype(MMD).reshape(cp * HWf, BB)
    s = jnp.dot(as_ref[...], xh, preferred_element_type=jnp.float32)
    feat = jnp.dot(wslt_ref[...], s.astype(MMD),
                   preferred_element_type=jnp.float32)
    feat = feat + bft_ref[...]
    prod = ce_ref[...] * feat
    o_ref[...] = jnp.sum(prod, axis=0, keepdims=True)[None]


def kernel(wslab, bslab, mslab, emb, img, c):
    B, cimg, H0, W0 = img.shape
    HW0 = H0 * W0
    C = emb.shape[1]
    xt = img.reshape(B, cimg, HW0).astype(MMD).transpose(1, 2, 0)
    cemb = emb[c].astype(jnp.float32).T

    mk0 = jnp.asarray(_tap_masks_pmajor(16, 16, BB, 8)).astype(MMD)
    mk1 = jnp.asarray(_tap_masks_pmajor(8, 8, BB, 16)).astype(MMD)
    mk2 = jnp.asarray(_tap_masks_pmajor(4, 4, BB, 16)).astype(MMD)
    p0 = jnp.asarray(np.kron(np.eye(POOL_CFG[0]['CG'], dtype=np.float32),
                             _pool_matrix_T(16, 16).T)).astype(MMD)
    p1 = jnp.asarray(np.kron(np.eye(POOL_CFG[1]['CG'], dtype=np.float32),
                             _pool_matrix_T(8, 8).T)).astype(MMD)

    hd = HEAD
    cp, HWf = hd['cp'], hd['hwf']
    wbig = wslab[hd['wbig_off']: hd['wbig_off'] + HWf, 0:hd['kcols']]
    a_s = jnp.zeros((cp * C, cp * HWf), MMD)
    for ci in range(cp):
        blk = wbig[:, ci * C:(ci + 1) * C].T
        a_s = a_s.at[ci * C:(ci + 1) * C, ci * HWf:(ci + 1) * HWf].set(blk)
    wslt = wslab[hd['wsl_off']: hd['wsl_off'] + hd['kcols'], 0:C].T
    bft = mslab[hd['bf_off']: hd['bf_off'] + 1, 0:C].T

    out = pl.pallas_call(
        _body,
        out_shape=jax.ShapeDtypeStruct((B // BB, 1, BB), jnp.float32),
        grid=(B // BB,),
        in_specs=[
            pl.BlockSpec((cimg, HW0, BB), lambda i: (0, 0, i)),
            pl.BlockSpec((C, BB), lambda i: (0, i)),
            pl.BlockSpec(wslab.shape, lambda i: (0, 0)),
            pl.BlockSpec(bslab.shape, lambda i: (0, 0)),
            pl.BlockSpec(mk0.shape, lambda i: (0, 0)),
            pl.BlockSpec(mk1.shape, lambda i: (0, 0)),
            pl.BlockSpec(mk2.shape, lambda i: (0, 0)),
            pl.BlockSpec(p0.shape, lambda i: (0, 0)),
            pl.BlockSpec(p1.shape, lambda i: (0, 0)),
            pl.BlockSpec(a_s.shape, lambda i: (0, 0)),
            pl.BlockSpec(wslt.shape, lambda i: (0, 0)),
            pl.BlockSpec(bft.shape, lambda i: (0, 0)),
        ],
        out_specs=pl.BlockSpec((1, 1, BB), lambda i: (i, 0, 0)),
        compiler_params=pltpu.CompilerParams(
            dimension_semantics=("parallel",)),
    )(xt, cemb, wslab, bslab, mk0, mk1, mk2, p0, p1, a_s, wslt, bft)
    return out.reshape(B, 1)


# speedup vs baseline: 70.8100x; 1.0895x over previous
import numpy as np
import jax
import jax.numpy as jnp
from jax.experimental import pallas as pl
from jax.experimental.pallas import tpu as pltpu

MMD = jnp.bfloat16
BB = 128

TAP_OFFS = ((-1, -1), (-1, 0), (-1, 1),
            (0, -1),  (0, 0),  (0, 1),
            (1, -1),  (1, 0),  (1, 1))

IN_META = dict(w_off=0, w_rows=8, w_cols=3, b_col=0)
BLOCKS = (
    dict(H=16, W=16, cpi=8,  cpo=8,  proj=True,  down=False, w_off=8,
         w_rows=16, w_cols=72,  b3=1, bp=2, m=0, mstride=8),
    dict(H=16, W=16, cpi=8,  cpo=8,  proj=False, down=True,  w_off=24,
         w_rows=8,  w_cols=72,  b3=3, m=0, pool=0, mstride=8),
    dict(H=8,  W=8,  cpi=8,  cpo=16, proj=True,  down=False, w_off=288,
         w_rows=32, w_cols=72,  b3=4, bp=5, m=1, mstride=16),
    dict(H=8,  W=8,  cpi=16, cpo=16, proj=False, down=True,  w_off=320,
         w_rows=16, w_cols=144, b3=6, m=1, pool=1, mstride=16),
    dict(H=4,  W=4,  cpi=16, cpo=16, proj=False, down=False, w_off=400,
         w_rows=16, w_cols=144, b3=7, m=2, mstride=16),
    dict(H=4,  W=4,  cpi=16, cpo=16, proj=False, down=False, w_off=416,
         w_rows=16, w_cols=144, b3=8, m=2, mstride=16),
)
POOL_CFG = ({'CG': 2, 'NG': 4}, {'CG': 8, 'NG': 2})
HEAD = dict(hwf=16, cp=16, c=16, kcols=256, wbig_off=432, wsl_off=448,
            dm_off=24, bf_off=40)


def _col_masks_pmajor(H, W, bb, ch):
    w = np.tile(np.arange(W), H)
    mL = (w > 0).astype(np.float32)[None, :]
    mR = (w < W - 1).astype(np.float32)[None, :]
    m = np.concatenate([np.repeat(mL, ch, 0), np.repeat(mR, ch, 0)], axis=0)
    return np.repeat(m, bb, axis=1)


def _pool_matrix_T(H, W):
    Ho, Wo = H // 2, W // 2
    P = np.zeros((H * W, Ho * Wo), np.float32)
    for i in range(Ho):
        for j in range(Wo):
            q = i * Wo + j
            for di in range(2):
                for dj in range(2):
                    P[(2 * i + di) * W + (2 * j + dj), q] = 0.25
    return P


def _resblock(x, km, w_ref, b_ref, mask, pools):
    H, W = km['H'], km['W']
    HW = H * W
    L = HW * BB
    cpo = km['cpo']

    cpi, mstride = km['cpi'], km['mstride']
    x16 = x.astype(MMD)
    cL = (jnp.concatenate([x16[:, L - BB:], x16[:, :L - BB]], axis=1)
          * mask[0:cpi, :])
    cR = (jnp.concatenate([x16[:, BB:], x16[:, :BB]], axis=1)
          * mask[mstride:mstride + cpi, :])
    zW = jnp.zeros((cpi, W * BB), MMD)
    bases = {-1: cL, 0: x16, 1: cR}

    def rowshift(b, dy):
        if dy == 0:
            return b
        if dy > 0:
            return jnp.concatenate([b[:, W * BB:], zW], axis=1)
        return jnp.concatenate([zW, b[:, :L - W * BB]], axis=1)

    taps = [rowshift(bases[dx], dy) for (dy, dx) in TAP_OFFS]
    xcol = jnp.concatenate(taps, axis=0)

    Wc = w_ref[km['w_off']: km['w_off'] + km['w_rows'], 0:km['w_cols']]
    y = jnp.dot(Wc, xcol, preferred_element_type=jnp.float32)

    b3 = b_ref[0:cpo, km['b3']: km['b3'] + 1]
    if km['proj']:
        r = y[:cpo] + b3
        s = y[cpo:2 * cpo] + b_ref[0:cpo, km['bp']: km['bp'] + 1]
    else:
        r = y + b3
        s = x
    r = jnp.maximum(r, 0.2 * r)
    out = r + s

    if 'pool' in km:
        cfg = POOL_CFG[km['pool']]
        CG, NG = cfg['CG'], cfg['NG']
        Pg = pools[km['pool']][...]
        t = out.astype(MMD).reshape(cpo * HW, BB)
        rows = CG * HW
        parts = [jnp.dot(Pg, t[g * rows:(g + 1) * rows, :],
                         preferred_element_type=jnp.float32)
                 for g in range(NG)]
        y2 = jnp.concatenate(parts, axis=0)
        out = y2.reshape(cpo, (HW // 4) * BB)
    return out


def _body(xt_ref, ce_ref, w_ref, b_ref, mk0, mk1, mk2,
          p0_ref, p1_ref, as_ref, wslt_ref, bft_ref, o_ref):
    inm = IN_META
    Win = w_ref[inm['w_off']: inm['w_off'] + inm['w_rows'], 0:inm['w_cols']]
    xb = xt_ref[...].astype(MMD)
    xin = xb.T.reshape(inm['w_cols'], 256 * BB)
    x = (jnp.dot(Win, xin, preferred_element_type=jnp.float32)
         + b_ref[0:inm['w_rows'], inm['b_col']: inm['b_col'] + 1])

    masks = (mk0, mk0, mk1, mk1, mk2, mk2)
    pools = (p0_ref, p1_ref)
    for km, mref in zip(BLOCKS, masks):
        x = _resblock(x, km, w_ref, b_ref, mref[...], pools)

    hd = HEAD
    cp, C, HWf = hd['cp'], hd['c'], hd['hwf']
    xh = x.astype(MMD).reshape(cp * HWf, BB)
    s = jnp.dot(as_ref[...], xh, preferred_element_type=jnp.float32)
    feat = jnp.dot(wslt_ref[...], s.astype(MMD),
                   preferred_element_type=jnp.float32)
    feat = feat + bft_ref[...]
    prod = ce_ref[...] * feat
    o_ref[...] = jnp.sum(prod, axis=0, keepdims=True)[None]


def kernel(wslab, bslab, mslab, emb, img, c):
    B, cimg, H0, W0 = img.shape
    HW0 = H0 * W0
    C = emb.shape[1]
    xt = img.reshape(B, cimg * HW0)
    cemb = emb[c].astype(jnp.float32).T

    mk0 = jnp.asarray(_col_masks_pmajor(16, 16, BB, 8)).astype(MMD)
    mk1 = jnp.asarray(_col_masks_pmajor(8, 8, BB, 16)).astype(MMD)
    mk2 = jnp.asarray(_col_masks_pmajor(4, 4, BB, 16)).astype(MMD)
    p0 = jnp.asarray(np.kron(np.eye(POOL_CFG[0]['CG'], dtype=np.float32),
                             _pool_matrix_T(16, 16).T)).astype(MMD)
    p1 = jnp.asarray(np.kron(np.eye(POOL_CFG[1]['CG'], dtype=np.float32),
                             _pool_matrix_T(8, 8).T)).astype(MMD)

    hd = HEAD
    cp, HWf = hd['cp'], hd['hwf']
    wbig = wslab[hd['wbig_off']: hd['wbig_off'] + HWf, 0:hd['kcols']]
    a_s = jnp.zeros((cp * C, cp * HWf), MMD)
    for ci in range(cp):
        blk = wbig[:, ci * C:(ci + 1) * C].T
        a_s = a_s.at[ci * C:(ci + 1) * C, ci * HWf:(ci + 1) * HWf].set(blk)
    wslt = wslab[hd['wsl_off']: hd['wsl_off'] + hd['kcols'], 0:C].T
    bft = mslab[hd['bf_off']: hd['bf_off'] + 1, 0:C].T

    out = pl.pallas_call(
        _body,
        out_shape=jax.ShapeDtypeStruct((B // BB, 1, BB), jnp.float32),
        grid=(B // BB,),
        in_specs=[
            pl.BlockSpec((BB, cimg * HW0), lambda i: (i, 0)),
            pl.BlockSpec((C, BB), lambda i: (0, i)),
            pl.BlockSpec(wslab.shape, lambda i: (0, 0)),
            pl.BlockSpec(bslab.shape, lambda i: (0, 0)),
            pl.BlockSpec(mk0.shape, lambda i: (0, 0)),
            pl.BlockSpec(mk1.shape, lambda i: (0, 0)),
            pl.BlockSpec(mk2.shape, lambda i: (0, 0)),
            pl.BlockSpec(p0.shape, lambda i: (0, 0)),
            pl.BlockSpec(p1.shape, lambda i: (0, 0)),
            pl.BlockSpec(a_s.shape, lambda i: (0, 0)),
            pl.BlockSpec(wslt.shape, lambda i: (0, 0)),
            pl.BlockSpec(bft.shape, lambda i: (0, 0)),
        ],
        out_specs=pl.BlockSpec((1, 1, BB), lambda i: (i, 0, 0)),
        compiler_params=pltpu.CompilerParams(
            dimension_semantics=("parallel",)),
    )(xt, cemb, wslab, bslab, mk0, mk1, mk2, p0, p1, a_s, wslt, bft)
    return out.reshape(B, 1)


# speedup vs baseline: 78.0686x; 1.1025x over previous
import numpy as np
import jax
import jax.numpy as jnp
from jax.experimental import pallas as pl
from jax.experimental.pallas import tpu as pltpu

MMD = jnp.bfloat16
BB = 128

TAP_OFFS = ((-1, -1), (-1, 0), (-1, 1),
            (0, -1),  (0, 0),  (0, 1),
            (1, -1),  (1, 0),  (1, 1))

IN_META = dict(w_off=0, w_rows=8, w_cols=3, b_col=0)
BLOCKS = (
    dict(H=16, W=16, cpi=8,  cpo=8,  proj=True,  down=False, w_off=8,
         w_rows=16, w_cols=72,  b3=1, bp=2, m=0, mstride=8),
    dict(H=16, W=16, cpi=8,  cpo=8,  proj=False, down=True,  w_off=24,
         w_rows=8,  w_cols=72,  b3=3, m=0, pool=0, mstride=8),
    dict(H=8,  W=8,  cpi=8,  cpo=16, proj=True,  down=False, w_off=288,
         w_rows=32, w_cols=72,  b3=4, bp=5, m=1, mstride=16),
    dict(H=8,  W=8,  cpi=16, cpo=16, proj=False, down=True,  w_off=320,
         w_rows=16, w_cols=144, b3=6, m=1, pool=1, mstride=16),
    dict(H=4,  W=4,  cpi=16, cpo=16, proj=False, down=False, w_off=400,
         w_rows=16, w_cols=144, b3=7, m=2, mstride=16),
    dict(H=4,  W=4,  cpi=16, cpo=16, proj=False, down=False, w_off=416,
         w_rows=16, w_cols=144, b3=8, m=2, mstride=16),
)
POOL_CFG = ({'CG': 2, 'NG': 4}, {'CG': 8, 'NG': 2})
HEAD = dict(hwf=16, cp=16, c=16, kcols=256, wbig_off=432, wsl_off=448,
            dm_off=24, bf_off=40)


def _col_masks_pmajor(H, W, bb, ch):
    w = np.tile(np.arange(W), H)
    mL = (w > 0).astype(np.float32)[None, :]
    mR = (w < W - 1).astype(np.float32)[None, :]
    m = np.concatenate([np.repeat(mL, ch, 0), np.repeat(mR, ch, 0)], axis=0)
    return np.repeat(m, bb, axis=1)


def _pool_matrix_T(H, W):
    Ho, Wo = H // 2, W // 2
    P = np.zeros((H * W, Ho * Wo), np.float32)
    for i in range(Ho):
        for j in range(Wo):
            q = i * Wo + j
            for di in range(2):
                for dj in range(2):
                    P[(2 * i + di) * W + (2 * j + dj), q] = 0.25
    return P


def _resblock(x, km, w_ref, b_ref, mask, pools):
    H, W = km['H'], km['W']
    HW = H * W
    L = HW * BB
    cpo = km['cpo']

    cpi, mstride = km['cpi'], km['mstride']
    x16 = x.astype(MMD)
    cL = (jnp.concatenate([x16[:, L - BB:], x16[:, :L - BB]], axis=1)
          * mask[0:cpi, :])
    cR = (jnp.concatenate([x16[:, BB:], x16[:, :BB]], axis=1)
          * mask[mstride:mstride + cpi, :])
    zW = jnp.zeros((cpi, W * BB), MMD)
    bases = {-1: cL, 0: x16, 1: cR}

    def rowshift(b, dy):
        if dy == 0:
            return b
        if dy > 0:
            return jnp.concatenate([b[:, W * BB:], zW], axis=1)
        return jnp.concatenate([zW, b[:, :L - W * BB]], axis=1)

    taps = [rowshift(bases[dx], dy) for (dy, dx) in TAP_OFFS]
    xcol = jnp.concatenate(taps, axis=0)

    Wc = w_ref[km['w_off']: km['w_off'] + km['w_rows'], 0:km['w_cols']]
    y = jnp.dot(Wc, xcol, preferred_element_type=jnp.float32)

    b3 = b_ref[0:cpo, km['b3']: km['b3'] + 1]
    if km['proj']:
        r = y[:cpo] + b3
        s = y[cpo:2 * cpo] + b_ref[0:cpo, km['bp']: km['bp'] + 1]
    else:
        r = y + b3
        s = x
    r = jnp.maximum(r, 0.2 * r)
    out = r + s

    if 'pool' in km:
        cfg = POOL_CFG[km['pool']]
        CG, NG = cfg['CG'], cfg['NG']
        Pg = pools[km['pool']][...]
        t = out.astype(MMD).reshape(cpo * HW, BB)
        rows = CG * HW
        parts = [jnp.dot(Pg, t[g * rows:(g + 1) * rows, :],
                         preferred_element_type=jnp.float32)
                 for g in range(NG)]
        y2 = jnp.concatenate(parts, axis=0)
        out = y2.reshape(cpo, (HW // 4) * BB)
    return out


def _body(xt_ref, ce_ref, w_ref, b_ref, mk0, mk1, mk2,
          p0_ref, p1_ref, as_ref, wslt_ref, bft_ref, o_ref):
    inm = IN_META
    Win = w_ref[inm['w_off']: inm['w_off'] + inm['w_rows'], 0:inm['w_cols']]
    xb = xt_ref[...].astype(MMD)
    xin = xb.T.reshape(inm['w_cols'], 256 * BB)
    x = (jnp.dot(Win, xin, preferred_element_type=jnp.float32)
         + b_ref[0:inm['w_rows'], inm['b_col']: inm['b_col'] + 1])

    masks = (mk0, mk0, mk1, mk1, mk2, mk2)
    pools = (p0_ref, p1_ref)
    for km, mref in zip(BLOCKS, masks):
        x = _resblock(x, km, w_ref, b_ref, mref[...], pools)

    hd = HEAD
    cp, C, HWf = hd['cp'], hd['c'], hd['hwf']
    xh = x.astype(MMD).reshape(cp * HWf, BB)
    s = jnp.dot(as_ref[...], xh, preferred_element_type=jnp.float32)
    feat = jnp.dot(wslt_ref[...], s.astype(MMD),
                   preferred_element_type=jnp.float32)
    feat = feat + bft_ref[...]
    prod = ce_ref[...] * feat
    o_ref[...] = jnp.sum(prod, axis=0, keepdims=True)[None]


def kernel(wslab, bslab, mslab, emb, img, c):
    B, cimg, H0, W0 = img.shape
    HW0 = H0 * W0
    C = emb.shape[1]
    xt = img.reshape(B, cimg * HW0)
    oh = (c[None, :] == jnp.arange(emb.shape[0])[:, None]).astype(jnp.float32)
    cemb = emb.T.astype(jnp.float32) @ oh

    mk0 = jnp.asarray(_col_masks_pmajor(16, 16, BB, 8)).astype(MMD)
    mk1 = jnp.asarray(_col_masks_pmajor(8, 8, BB, 16)).astype(MMD)
    mk2 = jnp.asarray(_col_masks_pmajor(4, 4, BB, 16)).astype(MMD)
    p0 = jnp.asarray(np.kron(np.eye(POOL_CFG[0]['CG'], dtype=np.float32),
                             _pool_matrix_T(16, 16).T)).astype(MMD)
    p1 = jnp.asarray(np.kron(np.eye(POOL_CFG[1]['CG'], dtype=np.float32),
                             _pool_matrix_T(8, 8).T)).astype(MMD)

    hd = HEAD
    cp, HWf = hd['cp'], hd['hwf']
    wbig = wslab[hd['wbig_off']: hd['wbig_off'] + HWf, 0:hd['kcols']]
    a_s = jnp.zeros((cp * C, cp * HWf), MMD)
    for ci in range(cp):
        blk = wbig[:, ci * C:(ci + 1) * C].T
        a_s = a_s.at[ci * C:(ci + 1) * C, ci * HWf:(ci + 1) * HWf].set(blk)
    wslt = wslab[hd['wsl_off']: hd['wsl_off'] + hd['kcols'], 0:C].T
    bft = mslab[hd['bf_off']: hd['bf_off'] + 1, 0:C].T

    out = pl.pallas_call(
        _body,
        out_shape=jax.ShapeDtypeStruct((B // BB, 1, BB), jnp.float32),
        grid=(B // BB,),
        in_specs=[
            pl.BlockSpec((BB, cimg * HW0), lambda i: (i, 0)),
            pl.BlockSpec((C, BB), lambda i: (0, i)),
            pl.BlockSpec(wslab.shape, lambda i: (0, 0)),
            pl.BlockSpec(bslab.shape, lambda i: (0, 0)),
            pl.BlockSpec(mk0.shape, lambda i: (0, 0)),
            pl.BlockSpec(mk1.shape, lambda i: (0, 0)),
            pl.BlockSpec(mk2.shape, lambda i: (0, 0)),
            pl.BlockSpec(p0.shape, lambda i: (0, 0)),
            pl.BlockSpec(p1.shape, lambda i: (0, 0)),
            pl.BlockSpec(a_s.shape, lambda i: (0, 0)),
            pl.BlockSpec(wslt.shape, lambda i: (0, 0)),
            pl.BlockSpec(bft.shape, lambda i: (0, 0)),
        ],
        out_specs=pl.BlockSpec((1, 1, BB), lambda i: (i, 0, 0)),
        compiler_params=pltpu.CompilerParams(
            dimension_semantics=("parallel",)),
    )(xt, cemb, wslab, bslab, mk0, mk1, mk2, p0, p1, a_s, wslt, bft)
    return out.reshape(B, 1)


# speedup vs baseline: 78.1306x; 1.0008x over previous
import numpy as np
import jax
import jax.numpy as jnp
from jax.experimental import pallas as pl
from jax.experimental.pallas import tpu as pltpu

MMD = jnp.bfloat16
NCLS = 10
BB = 128

TAP_OFFS = ((-1, -1), (-1, 0), (-1, 1),
            (0, -1),  (0, 0),  (0, 1),
            (1, -1),  (1, 0),  (1, 1))

IN_META = dict(w_off=0, w_rows=8, w_cols=3, b_col=0)
BLOCKS = (
    dict(H=16, W=16, cpi=8,  cpo=8,  proj=True,  down=False, w_off=8,
         w_rows=16, w_cols=72,  b3=1, bp=2, m=0, mstride=8),
    dict(H=16, W=16, cpi=8,  cpo=8,  proj=False, down=True,  w_off=24,
         w_rows=8,  w_cols=72,  b3=3, m=0, pool=0, mstride=8),
    dict(H=8,  W=8,  cpi=8,  cpo=16, proj=True,  down=False, w_off=288,
         w_rows=32, w_cols=72,  b3=4, bp=5, m=1, mstride=16),
    dict(H=8,  W=8,  cpi=16, cpo=16, proj=False, down=True,  w_off=320,
         w_rows=16, w_cols=144, b3=6, m=1, pool=1, mstride=16),
    dict(H=4,  W=4,  cpi=16, cpo=16, proj=False, down=False, w_off=400,
         w_rows=16, w_cols=144, b3=7, m=2, mstride=16),
    dict(H=4,  W=4,  cpi=16, cpo=16, proj=False, down=False, w_off=416,
         w_rows=16, w_cols=144, b3=8, m=2, mstride=16),
)
POOL_CFG = ({'CG': 2, 'NG': 4}, {'CG': 8, 'NG': 2})
HEAD = dict(hwf=16, cp=16, c=16, kcols=256, wbig_off=432, wsl_off=448,
            dm_off=24, bf_off=40)


def _col_masks_pmajor(H, W, bb, ch):
    w = np.tile(np.arange(W), H)
    mL = (w > 0).astype(np.float32)[None, :]
    mR = (w < W - 1).astype(np.float32)[None, :]
    m = np.concatenate([np.repeat(mL, ch, 0), np.repeat(mR, ch, 0)], axis=0)
    return np.repeat(m, bb, axis=1)


def _pool_matrix_T(H, W):
    Ho, Wo = H // 2, W // 2
    P = np.zeros((H * W, Ho * Wo), np.float32)
    for i in range(Ho):
        for j in range(Wo):
            q = i * Wo + j
            for di in range(2):
                for dj in range(2):
                    P[(2 * i + di) * W + (2 * j + dj), q] = 0.25
    return P


def _resblock(x, km, w_ref, b_ref, mask, pools):
    H, W = km['H'], km['W']
    HW = H * W
    L = HW * BB
    cpo = km['cpo']

    cpi, mstride = km['cpi'], km['mstride']
    x16 = x.astype(MMD)
    cL = (jnp.concatenate([x16[:, L - BB:], x16[:, :L - BB]], axis=1)
          * mask[0:cpi, :])
    cR = (jnp.concatenate([x16[:, BB:], x16[:, :BB]], axis=1)
          * mask[mstride:mstride + cpi, :])
    zW = jnp.zeros((cpi, W * BB), MMD)
    bases = {-1: cL, 0: x16, 1: cR}

    def rowshift(b, dy):
        if dy == 0:
            return b
        if dy > 0:
            return jnp.concatenate([b[:, W * BB:], zW], axis=1)
        return jnp.concatenate([zW, b[:, :L - W * BB]], axis=1)

    taps = [rowshift(bases[dx], dy) for (dy, dx) in TAP_OFFS]
    xcol = jnp.concatenate(taps, axis=0)

    Wc = w_ref[km['w_off']: km['w_off'] + km['w_rows'], 0:km['w_cols']]
    y = jnp.dot(Wc, xcol, preferred_element_type=jnp.float32)

    b3 = b_ref[0:cpo, km['b3']: km['b3'] + 1]
    if km['proj']:
        r = y[:cpo] + b3
        s = y[cpo:2 * cpo] + b_ref[0:cpo, km['bp']: km['bp'] + 1]
    else:
        r = y + b3
        s = x
    r = jnp.maximum(r, 0.2 * r)
    out = r + s

    if 'pool' in km:
        cfg = POOL_CFG[km['pool']]
        CG, NG = cfg['CG'], cfg['NG']
        Pg = pools[km['pool']][...]
        t = out.astype(MMD).reshape(cpo * HW, BB)
        rows = CG * HW
        parts = [jnp.dot(Pg, t[g * rows:(g + 1) * rows, :],
                         preferred_element_type=jnp.float32)
                 for g in range(NG)]
        y2 = jnp.concatenate(parts, axis=0)
        out = y2.reshape(cpo, (HW // 4) * BB)
    return out


def _body(xt_ref, ce_ref, emb_ref, w_ref, b_ref, mk0, mk1, mk2,
          p0_ref, p1_ref, as_ref, wslt_ref, bft_ref, o_ref):
    inm = IN_META
    Win = w_ref[inm['w_off']: inm['w_off'] + inm['w_rows'], 0:inm['w_cols']]
    xb = xt_ref[...].astype(MMD)
    xin = xb.T.reshape(inm['w_cols'], 256 * BB)
    x = (jnp.dot(Win, xin, preferred_element_type=jnp.float32)
         + b_ref[0:inm['w_rows'], inm['b_col']: inm['b_col'] + 1])

    masks = (mk0, mk0, mk1, mk1, mk2, mk2)
    pools = (p0_ref, p1_ref)
    for km, mref in zip(BLOCKS, masks):
        x = _resblock(x, km, w_ref, b_ref, mref[...], pools)

    hd = HEAD
    cp, C, HWf = hd['cp'], hd['c'], hd['hwf']
    xh = x.astype(MMD).reshape(cp * HWf, BB)
    s = jnp.dot(as_ref[...], xh, preferred_element_type=jnp.float32)
    feat = jnp.dot(wslt_ref[...], s.astype(MMD),
                   preferred_element_type=jnp.float32)
    feat = feat + bft_ref[...]
    cls = jax.lax.broadcasted_iota(jnp.int32, (NCLS * C, BB), 0) // C
    ce = jnp.where(cls == ce_ref[0], emb_ref[...], 0.0)
    cemb = ce[0:C, :]
    for k in range(1, NCLS):
        cemb = cemb + ce[k * C:(k + 1) * C, :]
    prod = cemb * feat
    o_ref[...] = jnp.sum(prod, axis=0, keepdims=True)[None]


def kernel(wslab, bslab, mslab, emb, img, c):
    B, cimg, H0, W0 = img.shape
    HW0 = H0 * W0
    C = emb.shape[1]
    xt = img.reshape(B, cimg * HW0)
    carr = c.astype(jnp.int32).reshape(B // BB, 1, BB)
    embb = jnp.tile(emb.astype(jnp.float32).reshape(NCLS * C, 1), (1, BB))

    mk0 = jnp.asarray(_col_masks_pmajor(16, 16, BB, 8)).astype(MMD)
    mk1 = jnp.asarray(_col_masks_pmajor(8, 8, BB, 16)).astype(MMD)
    mk2 = jnp.asarray(_col_masks_pmajor(4, 4, BB, 16)).astype(MMD)
    p0 = jnp.asarray(np.kron(np.eye(POOL_CFG[0]['CG'], dtype=np.float32),
                             _pool_matrix_T(16, 16).T)).astype(MMD)
    p1 = jnp.asarray(np.kron(np.eye(POOL_CFG[1]['CG'], dtype=np.float32),
                             _pool_matrix_T(8, 8).T)).astype(MMD)

    hd = HEAD
    cp, HWf = hd['cp'], hd['hwf']
    wbig = wslab[hd['wbig_off']: hd['wbig_off'] + HWf, 0:hd['kcols']]
    a_s = jnp.zeros((cp * C, cp * HWf), MMD)
    for ci in range(cp):
        blk = wbig[:, ci * C:(ci + 1) * C].T
        a_s = a_s.at[ci * C:(ci + 1) * C, ci * HWf:(ci + 1) * HWf].set(blk)
    wslt = wslab[hd['wsl_off']: hd['wsl_off'] + hd['kcols'], 0:C].T
    bft = mslab[hd['bf_off']: hd['bf_off'] + 1, 0:C].T

    out = pl.pallas_call(
        _body,
        out_shape=jax.ShapeDtypeStruct((B // BB, 1, BB), jnp.float32),
        grid=(B // BB,),
        in_specs=[
            pl.BlockSpec((BB, cimg * HW0), lambda i: (i, 0)),
            pl.BlockSpec((1, 1, BB), lambda i: (i, 0, 0)),
            pl.BlockSpec(embb.shape, lambda i: (0, 0)),
            pl.BlockSpec(wslab.shape, lambda i: (0, 0)),
            pl.BlockSpec(bslab.shape, lambda i: (0, 0)),
            pl.BlockSpec(mk0.shape, lambda i: (0, 0)),
            pl.BlockSpec(mk1.shape, lambda i: (0, 0)),
            pl.BlockSpec(mk2.shape, lambda i: (0, 0)),
            pl.BlockSpec(p0.shape, lambda i: (0, 0)),
            pl.BlockSpec(p1.shape, lambda i: (0, 0)),
            pl.BlockSpec(a_s.shape, lambda i: (0, 0)),
            pl.BlockSpec(wslt.shape, lambda i: (0, 0)),
            pl.BlockSpec(bft.shape, lambda i: (0, 0)),
        ],
        out_specs=pl.BlockSpec((1, 1, BB), lambda i: (i, 0, 0)),
        compiler_params=pltpu.CompilerParams(
            dimension_semantics=("parallel",)),
    )(xt, carr, embb, wslab, bslab, mk0, mk1, mk2, p0, p1, a_s, wslt, bft)
    return out.reshape(B, 1)


# speedup vs baseline: 91.9855x; 1.1773x over previous
import numpy as np
import jax
import jax.numpy as jnp
from jax.experimental import pallas as pl
from jax.experimental.pallas import tpu as pltpu

MMD = jnp.bfloat16
NCLS = 10
BB = 256

TAP_OFFS = ((-1, -1), (-1, 0), (-1, 1),
            (0, -1),  (0, 0),  (0, 1),
            (1, -1),  (1, 0),  (1, 1))

IN_META = dict(w_off=0, w_rows=8, w_cols=3, b_col=0)
BLOCKS = (
    dict(H=16, W=16, cpi=8,  cpo=8,  proj=True,  down=False, w_off=8,
         w_rows=16, w_cols=72,  b3=1, bp=2, m=0, mstride=8),
    dict(H=16, W=16, cpi=8,  cpo=8,  proj=False, down=True,  w_off=24,
         w_rows=8,  w_cols=72,  b3=3, m=0, pool=0, mstride=8),
    dict(H=8,  W=8,  cpi=8,  cpo=16, proj=True,  down=False, w_off=288,
         w_rows=32, w_cols=72,  b3=4, bp=5, m=1, mstride=16),
    dict(H=8,  W=8,  cpi=16, cpo=16, proj=False, down=True,  w_off=320,
         w_rows=16, w_cols=144, b3=6, m=1, pool=1, mstride=16),
    dict(H=4,  W=4,  cpi=16, cpo=16, proj=False, down=False, w_off=400,
         w_rows=16, w_cols=144, b3=7, m=2, mstride=16),
    dict(H=4,  W=4,  cpi=16, cpo=16, proj=False, down=False, w_off=416,
         w_rows=16, w_cols=144, b3=8, m=2, mstride=16),
)
POOL_CFG = ({'CG': 2, 'NG': 4}, {'CG': 8, 'NG': 2})
HEAD = dict(hwf=16, cp=16, c=16, kcols=256, wbig_off=432, wsl_off=448,
            dm_off=24, bf_off=40)


def _col_masks_pmajor(H, W, bb, ch):
    w = np.tile(np.arange(W), H)
    mL = (w > 0).astype(np.float32)[None, :]
    mR = (w < W - 1).astype(np.float32)[None, :]
    m = np.concatenate([np.repeat(mL, ch, 0), np.repeat(mR, ch, 0)], axis=0)
    return np.repeat(m, bb, axis=1)


def _pool_matrix_T(H, W):
    Ho, Wo = H // 2, W // 2
    P = np.zeros((H * W, Ho * Wo), np.float32)
    for i in range(Ho):
        for j in range(Wo):
            q = i * Wo + j
            for di in range(2):
                for dj in range(2):
                    P[(2 * i + di) * W + (2 * j + dj), q] = 0.25
    return P


def _resblock(x, km, w_ref, b_ref, mask, pools):
    H, W = km['H'], km['W']
    HW = H * W
    L = HW * BB
    cpo = km['cpo']

    cpi, mstride = km['cpi'], km['mstride']
    x16 = x.astype(MMD)
    cL = (jnp.concatenate([x16[:, L - BB:], x16[:, :L - BB]], axis=1)
          * mask[0:cpi, :])
    cR = (jnp.concatenate([x16[:, BB:], x16[:, :BB]], axis=1)
          * mask[mstride:mstride + cpi, :])
    zW = jnp.zeros((cpi, W * BB), MMD)
    bases = {-1: cL, 0: x16, 1: cR}

    def rowshift(b, dy):
        if dy == 0:
            return b
        if dy > 0:
            return jnp.concatenate([b[:, W * BB:], zW], axis=1)
        return jnp.concatenate([zW, b[:, :L - W * BB]], axis=1)

    taps = [rowshift(bases[dx], dy) for (dy, dx) in TAP_OFFS]
    xcol = jnp.concatenate(taps, axis=0)

    Wc = w_ref[km['w_off']: km['w_off'] + km['w_rows'], 0:km['w_cols']]
    y = jnp.dot(Wc, xcol, preferred_element_type=jnp.float32)

    b3 = b_ref[0:cpo, km['b3']: km['b3'] + 1]
    if km['proj']:
        r = y[:cpo] + b3
        s = y[cpo:2 * cpo] + b_ref[0:cpo, km['bp']: km['bp'] + 1]
    else:
        r = y + b3
        s = x
    r = jnp.maximum(r, 0.2 * r)
    out = r + s

    if 'pool' in km:
        cfg = POOL_CFG[km['pool']]
        CG, NG = cfg['CG'], cfg['NG']
        Pg = pools[km['pool']][...]
        t = out.astype(MMD).reshape(cpo * HW, BB)
        rows = CG * HW
        parts = [jnp.dot(Pg, t[g * rows:(g + 1) * rows, :],
                         preferred_element_type=jnp.float32)
                 for g in range(NG)]
        y2 = jnp.concatenate(parts, axis=0)
        out = y2.reshape(cpo, (HW // 4) * BB)
    return out


def _body(xt_ref, ce_ref, emb_ref, w_ref, b_ref, mk0, mk1, mk2,
          p0_ref, p1_ref, as_ref, wslt_ref, bft_ref, o_ref):
    inm = IN_META
    Win = w_ref[inm['w_off']: inm['w_off'] + inm['w_rows'], 0:inm['w_cols']]
    xb = xt_ref[...].astype(MMD)
    xin = xb.T.reshape(inm['w_cols'], 256 * BB)
    x = (jnp.dot(Win, xin, preferred_element_type=jnp.float32)
         + b_ref[0:inm['w_rows'], inm['b_col']: inm['b_col'] + 1])

    masks = (mk0, mk0, mk1, mk1, mk2, mk2)
    pools = (p0_ref, p1_ref)
    for km, mref in zip(BLOCKS, masks):
        x = _resblock(x, km, w_ref, b_ref, mref[...], pools)

    hd = HEAD
    cp, C, HWf = hd['cp'], hd['c'], hd['hwf']
    xh = x.astype(MMD).reshape(cp * HWf, BB)
    s = jnp.dot(as_ref[...], xh, preferred_element_type=jnp.float32)
    feat = jnp.dot(wslt_ref[...], s.astype(MMD),
                   preferred_element_type=jnp.float32)
    feat = feat + bft_ref[...]
    cls = jax.lax.broadcasted_iota(jnp.int32, (NCLS * C, BB), 0) // C
    ce = jnp.where(cls == ce_ref[0], emb_ref[...], 0.0)
    cemb = ce[0:C, :]
    for k in range(1, NCLS):
        cemb = cemb + ce[k * C:(k + 1) * C, :]
    prod = cemb * feat
    o_ref[...] = jnp.sum(prod, axis=0, keepdims=True)[None]


def kernel(wslab, bslab, mslab, emb, img, c):
    B, cimg, H0, W0 = img.shape
    HW0 = H0 * W0
    C = emb.shape[1]
    xt = img.reshape(B, cimg * HW0)
    carr = c.astype(jnp.int32).reshape(B // BB, 1, BB)
    embb = jnp.tile(emb.astype(jnp.float32).reshape(NCLS * C, 1), (1, BB))

    mk0 = jnp.asarray(_col_masks_pmajor(16, 16, BB, 8)).astype(MMD)
    mk1 = jnp.asarray(_col_masks_pmajor(8, 8, BB, 16)).astype(MMD)
    mk2 = jnp.asarray(_col_masks_pmajor(4, 4, BB, 16)).astype(MMD)
    p0 = jnp.asarray(np.kron(np.eye(POOL_CFG[0]['CG'], dtype=np.float32),
                             _pool_matrix_T(16, 16).T)).astype(MMD)
    p1 = jnp.asarray(np.kron(np.eye(POOL_CFG[1]['CG'], dtype=np.float32),
                             _pool_matrix_T(8, 8).T)).astype(MMD)

    hd = HEAD
    cp, HWf = hd['cp'], hd['hwf']
    wbig = wslab[hd['wbig_off']: hd['wbig_off'] + HWf, 0:hd['kcols']]
    a_s = jnp.zeros((cp * C, cp * HWf), MMD)
    for ci in range(cp):
        blk = wbig[:, ci * C:(ci + 1) * C].T
        a_s = a_s.at[ci * C:(ci + 1) * C, ci * HWf:(ci + 1) * HWf].set(blk)
    wslt = wslab[hd['wsl_off']: hd['wsl_off'] + hd['kcols'], 0:C].T
    bft = mslab[hd['bf_off']: hd['bf_off'] + 1, 0:C].T

    out = pl.pallas_call(
        _body,
        out_shape=jax.ShapeDtypeStruct((B // BB, 1, BB), jnp.float32),
        grid=(B // BB,),
        in_specs=[
            pl.BlockSpec((BB, cimg * HW0), lambda i: (i, 0)),
            pl.BlockSpec((1, 1, BB), lambda i: (i, 0, 0)),
            pl.BlockSpec(embb.shape, lambda i: (0, 0)),
            pl.BlockSpec(wslab.shape, lambda i: (0, 0)),
            pl.BlockSpec(bslab.shape, lambda i: (0, 0)),
            pl.BlockSpec(mk0.shape, lambda i: (0, 0)),
            pl.BlockSpec(mk1.shape, lambda i: (0, 0)),
            pl.BlockSpec(mk2.shape, lambda i: (0, 0)),
            pl.BlockSpec(p0.shape, lambda i: (0, 0)),
            pl.BlockSpec(p1.shape, lambda i: (0, 0)),
            pl.BlockSpec(a_s.shape, lambda i: (0, 0)),
            pl.BlockSpec(wslt.shape, lambda i: (0, 0)),
            pl.BlockSpec(bft.shape, lambda i: (0, 0)),
        ],
        out_specs=pl.BlockSpec((1, 1, BB), lambda i: (i, 0, 0)),
        compiler_params=pltpu.CompilerParams(
            dimension_semantics=("parallel",)),
    )(xt, carr, embb, wslab, bslab, mk0, mk1, mk2, p0, p1, a_s, wslt, bft)
    return out.reshape(B, 1)


# speedup vs baseline: 96.7912x; 1.0522x over previous
import numpy as np
import jax
import jax.numpy as jnp
from jax.experimental import pallas as pl
from jax.experimental.pallas import tpu as pltpu

MMD = jnp.bfloat16
NCLS = 10
BB = 512

TAP_OFFS = ((-1, -1), (-1, 0), (-1, 1),
            (0, -1),  (0, 0),  (0, 1),
            (1, -1),  (1, 0),  (1, 1))

IN_META = dict(w_off=0, w_rows=8, w_cols=3, b_col=0)
BLOCKS = (
    dict(H=16, W=16, cpi=8,  cpo=8,  proj=True,  down=False, w_off=8,
         w_rows=16, w_cols=72,  b3=1, bp=2, m=0, mstride=8),
    dict(H=16, W=16, cpi=8,  cpo=8,  proj=False, down=True,  w_off=24,
         w_rows=8,  w_cols=72,  b3=3, m=0, pool=0, mstride=8),
    dict(H=8,  W=8,  cpi=8,  cpo=16, proj=True,  down=False, w_off=288,
         w_rows=32, w_cols=72,  b3=4, bp=5, m=1, mstride=16),
    dict(H=8,  W=8,  cpi=16, cpo=16, proj=False, down=True,  w_off=320,
         w_rows=16, w_cols=144, b3=6, m=1, pool=1, mstride=16),
    dict(H=4,  W=4,  cpi=16, cpo=16, proj=False, down=False, w_off=400,
         w_rows=16, w_cols=144, b3=7, m=2, mstride=16),
    dict(H=4,  W=4,  cpi=16, cpo=16, proj=False, down=False, w_off=416,
         w_rows=16, w_cols=144, b3=8, m=2, mstride=16),
)
POOL_CFG = ({'CG': 2, 'NG': 4}, {'CG': 8, 'NG': 2})
HEAD = dict(hwf=16, cp=16, c=16, kcols=256, wbig_off=432, wsl_off=448,
            dm_off=24, bf_off=40)


def _col_masks_pmajor(H, W, bb, ch):
    w = np.tile(np.arange(W), H)
    mL = (w > 0).astype(np.float32)[None, :]
    mR = (w < W - 1).astype(np.float32)[None, :]
    m = np.concatenate([np.repeat(mL, ch, 0), np.repeat(mR, ch, 0)], axis=0)
    return np.repeat(m, bb, axis=1)


def _pool_matrix_T(H, W):
    Ho, Wo = H // 2, W // 2
    P = np.zeros((H * W, Ho * Wo), np.float32)
    for i in range(Ho):
        for j in range(Wo):
            q = i * Wo + j
            for di in range(2):
                for dj in range(2):
                    P[(2 * i + di) * W + (2 * j + dj), q] = 0.25
    return P


def _resblock(x, km, w_ref, b_ref, mask, pools):
    H, W = km['H'], km['W']
    HW = H * W
    L = HW * BB
    cpo = km['cpo']

    cpi, mstride = km['cpi'], km['mstride']
    x16 = x.astype(MMD)
    cL = (jnp.concatenate([x16[:, L - BB:], x16[:, :L - BB]], axis=1)
          * mask[0:cpi, :])
    cR = (jnp.concatenate([x16[:, BB:], x16[:, :BB]], axis=1)
          * mask[mstride:mstride + cpi, :])
    zW = jnp.zeros((cpi, W * BB), MMD)
    bases = {-1: cL, 0: x16, 1: cR}

    def rowshift(b, dy):
        if dy == 0:
            return b
        if dy > 0:
            return jnp.concatenate([b[:, W * BB:], zW], axis=1)
        return jnp.concatenate([zW, b[:, :L - W * BB]], axis=1)

    taps = [rowshift(bases[dx], dy) for (dy, dx) in TAP_OFFS]
    xcol = jnp.concatenate(taps, axis=0)

    Wc = w_ref[km['w_off']: km['w_off'] + km['w_rows'], 0:km['w_cols']]
    y = jnp.dot(Wc, xcol, preferred_element_type=jnp.float32)

    b3 = b_ref[0:cpo, km['b3']: km['b3'] + 1]
    if km['proj']:
        r = y[:cpo] + b3
        s = y[cpo:2 * cpo] + b_ref[0:cpo, km['bp']: km['bp'] + 1]
    else:
        r = y + b3
        s = x
    r = jnp.maximum(r, 0.2 * r)
    out = r + s

    if 'pool' in km:
        cfg = POOL_CFG[km['pool']]
        CG, NG = cfg['CG'], cfg['NG']
        Pg = pools[km['pool']][...]
        t = out.astype(MMD).reshape(cpo * HW, BB)
        rows = CG * HW
        parts = [jnp.dot(Pg, t[g * rows:(g + 1) * rows, :],
                         preferred_element_type=jnp.float32)
                 for g in range(NG)]
        y2 = jnp.concatenate(parts, axis=0)
        out = y2.reshape(cpo, (HW // 4) * BB)
    return out


def _body(xt_ref, ce_ref, emb_ref, w_ref, b_ref, mk0, mk1, mk2,
          p0_ref, p1_ref, as_ref, wslt_ref, bft_ref, o_ref):
    inm = IN_META
    Win = w_ref[inm['w_off']: inm['w_off'] + inm['w_rows'], 0:inm['w_cols']]
    xb = xt_ref[...].astype(MMD)
    xin = xb.T.reshape(inm['w_cols'], 256 * BB)
    x = (jnp.dot(Win, xin, preferred_element_type=jnp.float32)
         + b_ref[0:inm['w_rows'], inm['b_col']: inm['b_col'] + 1])

    masks = (mk0, mk0, mk1, mk1, mk2, mk2)
    pools = (p0_ref, p1_ref)
    for km, mref in zip(BLOCKS, masks):
        x = _resblock(x, km, w_ref, b_ref, mref[...], pools)

    hd = HEAD
    cp, C, HWf = hd['cp'], hd['c'], hd['hwf']
    xh = x.astype(MMD).reshape(cp * HWf, BB)
    s = jnp.dot(as_ref[...], xh, preferred_element_type=jnp.float32)
    feat = jnp.dot(wslt_ref[...], s.astype(MMD),
                   preferred_element_type=jnp.float32)
    feat = feat + bft_ref[...]
    cls = jax.lax.broadcasted_iota(jnp.int32, (NCLS * C, BB), 0) // C
    ce = jnp.where(cls == ce_ref[0], emb_ref[...], 0.0)
    cemb = ce[0:C, :]
    for k in range(1, NCLS):
        cemb = cemb + ce[k * C:(k + 1) * C, :]
    prod = cemb * feat
    o_ref[...] = jnp.sum(prod, axis=0, keepdims=True)[None]


def kernel(wslab, bslab, mslab, emb, img, c):
    B, cimg, H0, W0 = img.shape
    HW0 = H0 * W0
    C = emb.shape[1]
    xt = img.reshape(B, cimg * HW0)
    carr = c.astype(jnp.int32).reshape(B // BB, 1, BB)
    embb = jnp.tile(emb.astype(jnp.float32).reshape(NCLS * C, 1), (1, BB))

    mk0 = jnp.asarray(_col_masks_pmajor(16, 16, BB, 8)).astype(MMD)
    mk1 = jnp.asarray(_col_masks_pmajor(8, 8, BB, 16)).astype(MMD)
    mk2 = jnp.asarray(_col_masks_pmajor(4, 4, BB, 16)).astype(MMD)
    p0 = jnp.asarray(np.kron(np.eye(POOL_CFG[0]['CG'], dtype=np.float32),
                             _pool_matrix_T(16, 16).T)).astype(MMD)
    p1 = jnp.asarray(np.kron(np.eye(POOL_CFG[1]['CG'], dtype=np.float32),
                             _pool_matrix_T(8, 8).T)).astype(MMD)

    hd = HEAD
    cp, HWf = hd['cp'], hd['hwf']
    wbig = wslab[hd['wbig_off']: hd['wbig_off'] + HWf, 0:hd['kcols']]
    a_s = jnp.zeros((cp * C, cp * HWf), MMD)
    for ci in range(cp):
        blk = wbig[:, ci * C:(ci + 1) * C].T
        a_s = a_s.at[ci * C:(ci + 1) * C, ci * HWf:(ci + 1) * HWf].set(blk)
    wslt = wslab[hd['wsl_off']: hd['wsl_off'] + hd['kcols'], 0:C].T
    bft = mslab[hd['bf_off']: hd['bf_off'] + 1, 0:C].T

    out = pl.pallas_call(
        _body,
        out_shape=jax.ShapeDtypeStruct((B // BB, 1, BB), jnp.float32),
        grid=(B // BB,),
        in_specs=[
            pl.BlockSpec((BB, cimg * HW0), lambda i: (i, 0)),
            pl.BlockSpec((1, 1, BB), lambda i: (i, 0, 0)),
            pl.BlockSpec(embb.shape, lambda i: (0, 0)),
            pl.BlockSpec(wslab.shape, lambda i: (0, 0)),
            pl.BlockSpec(bslab.shape, lambda i: (0, 0)),
            pl.BlockSpec(mk0.shape, lambda i: (0, 0)),
            pl.BlockSpec(mk1.shape, lambda i: (0, 0)),
            pl.BlockSpec(mk2.shape, lambda i: (0, 0)),
            pl.BlockSpec(p0.shape, lambda i: (0, 0)),
            pl.BlockSpec(p1.shape, lambda i: (0, 0)),
            pl.BlockSpec(a_s.shape, lambda i: (0, 0)),
            pl.BlockSpec(wslt.shape, lambda i: (0, 0)),
            pl.BlockSpec(bft.shape, lambda i: (0, 0)),
        ],
        out_specs=pl.BlockSpec((1, 1, BB), lambda i: (i, 0, 0)),
        compiler_params=pltpu.CompilerParams(
            dimension_semantics=("parallel",)),
    )(xt, carr, embb, wslab, bslab, mk0, mk1, mk2, p0, p1, a_s, wslt, bft)
    return out.reshape(B, 1)


# speedup vs baseline: 97.1140x; 1.0033x over previous
import numpy as np
import jax
import jax.numpy as jnp
from jax.experimental import pallas as pl
from jax.experimental.pallas import tpu as pltpu

MMD = jnp.bfloat16
NCLS = 10
BB = 512

TAP_OFFS = ((-1, -1), (-1, 0), (-1, 1),
            (0, -1),  (0, 0),  (0, 1),
            (1, -1),  (1, 0),  (1, 1))

IN_META = dict(w_off=0, w_rows=8, w_cols=3, b_col=0)
BLOCKS = (
    dict(H=16, W=16, cpi=8,  cpo=8,  proj=True,  down=False, w_off=8,
         w_rows=16, w_cols=72,  b3=1, bp=2, m=0, mstride=8),
    dict(H=16, W=16, cpi=8,  cpo=8,  proj=False, down=True,  w_off=24,
         w_rows=8,  w_cols=72,  b3=3, m=0, pool=0, mstride=8),
    dict(H=8,  W=8,  cpi=8,  cpo=16, proj=True,  down=False, w_off=288,
         w_rows=32, w_cols=72,  b3=4, bp=5, m=1, mstride=16),
    dict(H=8,  W=8,  cpi=16, cpo=16, proj=False, down=True,  w_off=320,
         w_rows=16, w_cols=144, b3=6, m=1, pool=1, mstride=16),
    dict(H=4,  W=4,  cpi=16, cpo=16, proj=False, down=False, w_off=400,
         w_rows=16, w_cols=144, b3=7, m=2, mstride=16),
    dict(H=4,  W=4,  cpi=16, cpo=16, proj=False, down=False, w_off=416,
         w_rows=16, w_cols=144, b3=8, m=2, mstride=16),
)
POOL_CFG = ({'CG': 2, 'NG': 4}, {'CG': 8, 'NG': 2})
HEAD = dict(hwf=16, cp=16, c=16, kcols=256, wbig_off=432, wsl_off=448,
            dm_off=24, bf_off=40)


def _col_masks_pmajor(H, W, bb, ch):
    w = np.tile(np.arange(W), H)
    mL = (w > 0).astype(np.float32)[None, :]
    mR = (w < W - 1).astype(np.float32)[None, :]
    m = np.concatenate([np.repeat(mL, ch, 0), np.repeat(mR, ch, 0)], axis=0)
    return np.repeat(m, bb, axis=1)


def _pool_matrix_T(H, W):
    Ho, Wo = H // 2, W // 2
    P = np.zeros((H * W, Ho * Wo), np.float32)
    for i in range(Ho):
        for j in range(Wo):
            q = i * Wo + j
            for di in range(2):
                for dj in range(2):
                    P[(2 * i + di) * W + (2 * j + dj), q] = 0.25
    return P


def _resblock(x, km, w_ref, b_ref, mask, pools):
    H, W = km['H'], km['W']
    HW = H * W
    L = HW * BB
    cpo = km['cpo']

    cpi, mstride = km['cpi'], km['mstride']
    x16 = x[0:cpi, :].astype(MMD)
    cL = (jnp.concatenate([x16[:, L - BB:], x16[:, :L - BB]], axis=1)
          * mask[0:cpi, :])
    cR = (jnp.concatenate([x16[:, BB:], x16[:, :BB]], axis=1)
          * mask[mstride:mstride + cpi, :])
    zW = jnp.zeros((cpi, W * BB), MMD)
    bases = {-1: cL, 0: x16, 1: cR}

    def rowshift(b, dy):
        if dy == 0:
            return b
        if dy > 0:
            return jnp.concatenate([b[:, W * BB:], zW], axis=1)
        return jnp.concatenate([zW, b[:, :L - W * BB]], axis=1)

    taps = [rowshift(bases[dx], dy) for (dy, dx) in TAP_OFFS]
    xcol = jnp.concatenate(taps, axis=0)

    Wc = w_ref[km['w_off']: km['w_off'] + km['w_rows'], 0:km['w_cols']]
    y = jnp.dot(Wc, xcol, preferred_element_type=jnp.float32)

    b3 = b_ref[0:cpo, km['b3']: km['b3'] + 1]
    if km['proj']:
        r = y[:cpo] + b3
        s = y[cpo:2 * cpo] + b_ref[0:cpo, km['bp']: km['bp'] + 1]
    else:
        r = y + b3
        s = x
    r = jnp.maximum(r, 0.2 * r)
    out = r + s

    if 'pool' in km:
        cfg = POOL_CFG[km['pool']]
        CG, NG = cfg['CG'], cfg['NG']
        Pg = pools[km['pool']][...]
        t = out.astype(MMD).reshape(cpo * HW, BB)
        rows = CG * HW
        parts = [jnp.dot(Pg, t[g * rows:(g + 1) * rows, :],
                         preferred_element_type=jnp.float32)
                 for g in range(NG)]
        y2 = jnp.concatenate(parts, axis=0)
        out = y2.reshape(cpo, (HW // 4) * BB)
    return out


def _body(xt_ref, ce_ref, emb_ref, w_ref, b_ref, mk0, mk1, mk2,
          p0_ref, p1_ref, as_ref, wslt_ref, bft_ref, o_ref):
    inm = IN_META
    Win = w_ref[inm['w_off']: inm['w_off'] + inm['w_rows'], 0:inm['w_cols']]
    xb = xt_ref[...].astype(MMD)
    xin = xb.T.reshape(inm['w_cols'], 256 * BB)
    x = (jnp.dot(Win, xin, preferred_element_type=jnp.float32)
         + b_ref[0:inm['w_rows'], inm['b_col']: inm['b_col'] + 1])

    masks = (mk0, mk0, mk1, mk1, mk2, mk2)
    pools = (p0_ref, p1_ref)
    for km, mref in zip(BLOCKS, masks):
        x = _resblock(x, km, w_ref, b_ref, mref[...], pools)

    hd = HEAD
    cp, C, HWf = hd['cp'], hd['c'], hd['hwf']
    xh = x.astype(MMD).reshape(cp * HWf, BB)
    s = jnp.dot(as_ref[...], xh, preferred_element_type=jnp.float32)
    feat = jnp.dot(wslt_ref[...], s.astype(MMD),
                   preferred_element_type=jnp.float32)
    feat = feat + bft_ref[...]
    cls = jax.lax.broadcasted_iota(jnp.int32, (NCLS * C, BB), 0) // C
    ce = jnp.where(cls == ce_ref[0], emb_ref[...], 0.0)
    cemb = ce[0:C, :]
    for k in range(1, NCLS):
        cemb = cemb + ce[k * C:(k + 1) * C, :]
    prod = cemb * feat
    o_ref[...] = jnp.sum(prod, axis=0, keepdims=True)[None]


def kernel(wslab, bslab, mslab, emb, img, c):
    B, cimg, H0, W0 = img.shape
    HW0 = H0 * W0
    C = emb.shape[1]
    xt = img.reshape(B, cimg * HW0)
    carr = c.astype(jnp.int32).reshape(B // BB, 1, BB)
    embb = jnp.tile(emb.astype(jnp.float32).reshape(NCLS * C, 1), (1, BB))

    mk0 = jnp.asarray(_col_masks_pmajor(16, 16, BB, 8)).astype(MMD)
    mk1 = jnp.asarray(_col_masks_pmajor(8, 8, BB, 16)).astype(MMD)
    mk2 = jnp.asarray(_col_masks_pmajor(4, 4, BB, 16)).astype(MMD)
    p0 = jnp.asarray(np.kron(np.eye(POOL_CFG[0]['CG'], dtype=np.float32),
                             _pool_matrix_T(16, 16).T)).astype(MMD)
    p1 = jnp.asarray(np.kron(np.eye(POOL_CFG[1]['CG'], dtype=np.float32),
                             _pool_matrix_T(8, 8).T)).astype(MMD)

    hd = HEAD
    cp, HWf = hd['cp'], hd['hwf']
    wbig = wslab[hd['wbig_off']: hd['wbig_off'] + HWf, 0:hd['kcols']]
    a_s = jnp.zeros((cp * C, cp * HWf), MMD)
    for ci in range(cp):
        blk = wbig[:, ci * C:(ci + 1) * C].T
        a_s = a_s.at[ci * C:(ci + 1) * C, ci * HWf:(ci + 1) * HWf].set(blk)
    wslt = wslab[hd['wsl_off']: hd['wsl_off'] + hd['kcols'], 0:C].T
    bft = mslab[hd['bf_off']: hd['bf_off'] + 1, 0:C].T

    out = pl.pallas_call(
        _body,
        out_shape=jax.ShapeDtypeStruct((B // BB, 1, BB), jnp.float32),
        grid=(B // BB,),
        in_specs=[
            pl.BlockSpec((BB, cimg * HW0), lambda i: (i, 0)),
            pl.BlockSpec((1, 1, BB), lambda i: (i, 0, 0)),
            pl.BlockSpec(embb.shape, lambda i: (0, 0)),
            pl.BlockSpec(wslab.shape, lambda i: (0, 0)),
            pl.BlockSpec(bslab.shape, lambda i: (0, 0)),
            pl.BlockSpec(mk0.shape, lambda i: (0, 0)),
            pl.BlockSpec(mk1.shape, lambda i: (0, 0)),
            pl.BlockSpec(mk2.shape, lambda i: (0, 0)),
            pl.BlockSpec(p0.shape, lambda i: (0, 0)),
            pl.BlockSpec(p1.shape, lambda i: (0, 0)),
            pl.BlockSpec(a_s.shape, lambda i: (0, 0)),
            pl.BlockSpec(wslt.shape, lambda i: (0, 0)),
            pl.BlockSpec(bft.shape, lambda i: (0, 0)),
        ],
        out_specs=pl.BlockSpec((1, 1, BB), lambda i: (i, 0, 0)),
        compiler_params=pltpu.CompilerParams(
            dimension_semantics=("parallel",)),
    )(xt, carr, embb, wslab, bslab, mk0, mk1, mk2, p0, p1, a_s, wslt, bft)
    return out.reshape(B, 1)
